# baseline (speedup 1.0000x reference)
import functools

import jax
import jax.numpy as jnp
from jax import lax
from jax.experimental import pallas as pl
from jax.experimental.pallas import tpu as pltpu
from jax.experimental.pallas import tpu_sc as plsc

_NUM_WORKERS = 32
_CHUNK_ROWS = 32
_LANES = 16
_UNROLL = 8


def kernel(x, emb_weight):
    batch, seq_len, d = x.shape
    total_rows = batch * seq_len
    rows_per_w = total_rows // _NUM_WORKERS
    n_chunks = rows_per_w // _CHUNK_ROWS
    groups_per_row = d // (_LANES * _UNROLL)

    x2 = x.reshape(total_rows, d)

    mesh = plsc.VectorSubcoreMesh(core_axis_name="c", subcore_axis_name="s")

    @functools.partial(
        pl.kernel,
        mesh=mesh,
        out_type=jax.ShapeDtypeStruct((total_rows, d), jnp.float32),
        scratch_types=[
            pltpu.VMEM((_CHUNK_ROWS, d), jnp.float32),
            pltpu.VMEM((_CHUNK_ROWS, d), jnp.float32),
            pltpu.SemaphoreType.DMA,
            pltpu.SemaphoreType.DMA,
        ],
    )
    def sc_add(x_hbm, emb_hbm, out_hbm, xbuf, ebuf, semx, seme):
        cid = lax.axis_index("c")
        sid = lax.axis_index("s")
        wid = sid * 2 + cid
        base_row = wid * rows_per_w
        emb_base_row = lax.rem(base_row, seq_len)

        def chunk_body(k, _):
            row = base_row + k * _CHUNK_ROWS
            erow = emb_base_row + k * _CHUNK_ROWS
            cpx = pltpu.async_copy(x_hbm.at[pl.ds(row, _CHUNK_ROWS)], xbuf, semx)
            cpe = pltpu.async_copy(emb_hbm.at[pl.ds(erow, _CHUNK_ROWS)], ebuf, seme)
            cpx.wait()
            cpe.wait()

            def row_body(r, _):
                def grp_body(g, _):
                    base = g * (_LANES * _UNROLL)
                    for u in range(_UNROLL):
                        sl = pl.ds(base + u * _LANES, _LANES)
                        xbuf[r, sl] = xbuf[r, sl] + ebuf[r, sl]
                    return 0

                lax.fori_loop(0, groups_per_row, grp_body, 0)
                return 0

            lax.fori_loop(0, _CHUNK_ROWS, row_body, 0)
            pltpu.sync_copy(xbuf, out_hbm.at[pl.ds(row, _CHUNK_ROWS)])
            return 0

        lax.fori_loop(0, n_chunks, chunk_body, 0)

    out2 = sc_add(x2, emb_weight)
    return out2.reshape(batch, seq_len, d)


# speedup vs baseline: 1.3201x; 1.3201x over previous
import functools

import jax
import jax.numpy as jnp
from jax import lax
from jax.experimental import pallas as pl
from jax.experimental.pallas import tpu as pltpu
from jax.experimental.pallas import tpu_sc as plsc

_NUM_WORKERS = 32
_CHUNK_ROWS = 16
_LANES = 16
_UNROLL = 8


def kernel(x, emb_weight):
    batch, seq_len, d = x.shape
    total_rows = batch * seq_len
    rows_per_w = total_rows // _NUM_WORKERS
    n_chunks = rows_per_w // _CHUNK_ROWS
    groups = d // (_LANES * _UNROLL)
    C = _CHUNK_ROWS

    x2 = x.reshape(total_rows, d)

    mesh = plsc.VectorSubcoreMesh(core_axis_name="c", subcore_axis_name="s")

    @functools.partial(
        pl.kernel,
        mesh=mesh,
        out_type=jax.ShapeDtypeStruct((total_rows, d), jnp.float32),
        scratch_types=[
            pltpu.VMEM((C, d), jnp.float32),
            pltpu.VMEM((C, d), jnp.float32),
            pltpu.VMEM((C, d), jnp.float32),
            pltpu.VMEM((C, d), jnp.float32),
            pltpu.VMEM((C, d), jnp.float32),
            pltpu.VMEM((C, d), jnp.float32),
            pltpu.SemaphoreType.DMA,
            pltpu.SemaphoreType.DMA,
            pltpu.SemaphoreType.DMA,
            pltpu.SemaphoreType.DMA,
            pltpu.SemaphoreType.DMA,
            pltpu.SemaphoreType.DMA,
        ],
    )
    def sc_add(x_hbm, emb_hbm, out_hbm, xb0, xb1, eb0, eb1, ob0, ob1,
               sx0, sx1, se0, se1, so0, so1):
        xb, eb, ob = (xb0, xb1), (eb0, eb1), (ob0, ob1)
        sx, se, so = (sx0, sx1), (se0, se1), (so0, so1)

        cid = lax.axis_index("c")
        sid = lax.axis_index("s")
        wid = sid * 2 + cid
        base_row = wid * rows_per_w
        emb_base_row = lax.rem(base_row, seq_len)

        def issue_in(k, b):
            row = base_row + k * C
            erow = emb_base_row + k * C
            pltpu.async_copy(x_hbm.at[pl.ds(row, C)], xb[b], sx[b])
            pltpu.async_copy(emb_hbm.at[pl.ds(erow, C)], eb[b], se[b])

        def wait_in(k, b):
            row = base_row + k * C
            erow = emb_base_row + k * C
            pltpu.make_async_copy(x_hbm.at[pl.ds(row, C)], xb[b], sx[b]).wait()
            pltpu.make_async_copy(emb_hbm.at[pl.ds(erow, C)], eb[b], se[b]).wait()

        def wait_out(k, b):
            row = base_row + k * C
            pltpu.make_async_copy(ob[b], out_hbm.at[pl.ds(row, C)], so[b]).wait()

        def compute(b):
            def row_body(r, _):
                def grp_body(g, _):
                    base = g * (_LANES * _UNROLL)
                    for u in range(_UNROLL):
                        sl = pl.ds(base + u * _LANES, _LANES)
                        ob[b][r, sl] = xb[b][r, sl] + eb[b][r, sl]
                    return 0

                return lax.fori_loop(0, groups, grp_body, 0)

            lax.fori_loop(0, C, row_body, 0)

        def issue_out(k, b):
            row = base_row + k * C
            pltpu.async_copy(ob[b], out_hbm.at[pl.ds(row, C)], so[b])

        def step(k, b, first, last):
            wait_in(k, b)
            if not first:
                wait_out(k - 2, b)
            compute(b)
            issue_out(k, b)
            if not last:
                issue_in(k + 2, b)

        issue_in(0, 0)
        issue_in(1, 1)
        step(0, 0, True, False)
        step(1, 1, True, False)

        def pair_body(i, _):
            k = 2 * i
            step(k, 0, False, False)
            step(k + 1, 1, False, False)
            return 0

        lax.fori_loop(1, n_chunks // 2 - 1, pair_body, 0)

        step(n_chunks - 2, 0, False, True)
        step(n_chunks - 1, 1, False, True)
        wait_out(n_chunks - 2, 0)
        wait_out(n_chunks - 1, 1)

    out2 = sc_add(x2, emb_weight)
    return out2.reshape(batch, seq_len, d)


# speedup vs baseline: 2.9508x; 2.2354x over previous
import functools

import jax
import jax.numpy as jnp
from jax import lax
from jax.experimental import pallas as pl
from jax.experimental.pallas import tpu as pltpu
from jax.experimental.pallas import tpu_sc as plsc

_NUM_WORKERS = 32
_CHUNK_ROWS = 16
_LANES = 16
_UNROLL = 8


def kernel(x, emb_weight):
    batch, seq_len, d = x.shape
    total_rows = batch * seq_len
    rows_per_w = total_rows // _NUM_WORKERS
    n_chunks = rows_per_w // _CHUNK_ROWS
    C = _CHUNK_ROWS
    chunk = C * d
    slices_per_chunk = chunk // (_LANES * _UNROLL)

    x2 = x.reshape(total_rows, d)

    mesh = plsc.VectorSubcoreMesh(core_axis_name="c", subcore_axis_name="s")

    @functools.partial(
        pl.kernel,
        mesh=mesh,
        out_type=jax.ShapeDtypeStruct((total_rows, d), jnp.float32),
        scratch_types=[
            pltpu.VMEM((chunk,), jnp.float32),
            pltpu.VMEM((chunk,), jnp.float32),
            pltpu.VMEM((chunk,), jnp.float32),
            pltpu.VMEM((chunk,), jnp.float32),
            pltpu.VMEM((chunk,), jnp.float32),
            pltpu.VMEM((chunk,), jnp.float32),
            pltpu.SemaphoreType.DMA,
            pltpu.SemaphoreType.DMA,
            pltpu.SemaphoreType.DMA,
            pltpu.SemaphoreType.DMA,
            pltpu.SemaphoreType.DMA,
            pltpu.SemaphoreType.DMA,
        ],
    )
    def sc_add(x_hbm, emb_hbm, out_hbm, xb0, xb1, eb0, eb1, ob0, ob1,
               sx0, sx1, se0, se1, so0, so1):
        xb, eb, ob = (xb0, xb1), (eb0, eb1), (ob0, ob1)
        sx, se, so = (sx0, sx1), (se0, se1), (so0, so1)

        cid = lax.axis_index("c")
        sid = lax.axis_index("s")
        wid = sid * 2 + cid
        base_row = wid * rows_per_w
        emb_base_row = lax.rem(base_row, seq_len)

        def issue_in(k, b):
            row = base_row + k * C
            erow = emb_base_row + k * C
            for r in range(C):
                dst = pl.ds(r * d, d)
                pltpu.async_copy(x_hbm.at[row + r], xb[b].at[dst], sx[b])
                pltpu.async_copy(emb_hbm.at[erow + r], eb[b].at[dst], se[b])

        def wait_in(b):
            for r in range(C):
                dst = pl.ds(r * d, d)
                pltpu.make_async_copy(x_hbm.at[base_row], xb[b].at[dst], sx[b]).wait()
                pltpu.make_async_copy(emb_hbm.at[emb_base_row], eb[b].at[dst], se[b]).wait()

        def wait_out(b):
            for r in range(C):
                src = pl.ds(r * d, d)
                pltpu.make_async_copy(ob[b].at[src], out_hbm.at[base_row], so[b]).wait()

        def compute(b):
            def add_body(i, _):
                base = i * (_LANES * _UNROLL)
                for u in range(_UNROLL):
                    sl = pl.ds(base + u * _LANES, _LANES)
                    ob[b][sl] = xb[b][sl] + eb[b][sl]
                return 0

            lax.fori_loop(0, slices_per_chunk, add_body, 0)

        def issue_out(k, b):
            row = base_row + k * C
            for r in range(C):
                src = pl.ds(r * d, d)
                pltpu.async_copy(ob[b].at[src], out_hbm.at[row + r], so[b])

        def step(k, b, first, last):
            wait_in(b)
            if not first:
                wait_out(b)
            compute(b)
            issue_out(k, b)
            if not last:
                issue_in(k + 2, b)

        issue_in(0, 0)
        issue_in(1, 1)
        step(0, 0, True, False)
        step(1, 1, True, False)

        def pair_body(i, _):
            k = 2 * i
            step(k, 0, False, False)
            step(k + 1, 1, False, False)
            return 0

        lax.fori_loop(1, n_chunks // 2 - 1, pair_body, 0)

        step(n_chunks - 2, 0, False, True)
        step(n_chunks - 1, 1, False, True)
        wait_out(0)
        wait_out(1)

    out2 = sc_add(x2, emb_weight)
    return out2.reshape(batch, seq_len, d)


# speedup vs baseline: 2.9815x; 1.0104x over previous
import functools

import jax
import jax.numpy as jnp
from jax import lax
from jax.experimental import pallas as pl
from jax.experimental.pallas import tpu as pltpu
from jax.experimental.pallas import tpu_sc as plsc

_NUM_WORKERS = 32
_CHUNK_ROWS = 16
_LANES = 16
_UNROLL = 16


def kernel(x, emb_weight):
    batch, seq_len, d = x.shape
    total_rows = batch * seq_len
    rows_per_w = total_rows // _NUM_WORKERS
    n_chunks = rows_per_w // _CHUNK_ROWS
    C = _CHUNK_ROWS
    chunk = C * d
    slices_per_chunk = chunk // (_LANES * _UNROLL)

    x2 = x.reshape(total_rows, d)
    dummy = jnp.zeros((chunk,), jnp.float32)

    mesh = plsc.VectorSubcoreMesh(core_axis_name="c", subcore_axis_name="s")

    @functools.partial(
        pl.kernel,
        mesh=mesh,
        out_type=jax.ShapeDtypeStruct((total_rows, d), jnp.float32),
        scratch_types=[
            pltpu.VMEM((chunk,), jnp.float32),
            pltpu.VMEM((chunk,), jnp.float32),
            pltpu.VMEM((chunk,), jnp.float32),
            pltpu.VMEM((chunk,), jnp.float32),
            pltpu.VMEM((chunk,), jnp.float32),
            pltpu.VMEM((chunk,), jnp.float32),
            pltpu.SemaphoreType.DMA,
            pltpu.SemaphoreType.DMA,
            pltpu.SemaphoreType.DMA,
            pltpu.SemaphoreType.DMA,
            pltpu.SemaphoreType.DMA,
            pltpu.SemaphoreType.DMA,
        ],
    )
    def sc_add(x_hbm, emb_hbm, dummy_hbm, out_hbm, xb0, xb1, eb0, eb1, ob0, ob1,
               sx0, sx1, se0, se1, so0, so1):
        xb, eb, ob = (xb0, xb1), (eb0, eb1), (ob0, ob1)
        sx, se, so = (sx0, sx1), (se0, se1), (so0, so1)

        cid = lax.axis_index("c")
        sid = lax.axis_index("s")
        wid = sid * 2 + cid
        base_row = wid * rows_per_w
        emb_base_row = lax.rem(base_row, seq_len)

        def issue_in(k, b):
            row = base_row + k * C
            erow = emb_base_row + k * C
            for r in range(C):
                dst = pl.ds(r * d, d)
                pltpu.async_copy(x_hbm.at[row + r], xb[b].at[dst], sx[b])
                pltpu.async_copy(emb_hbm.at[erow + r], eb[b].at[dst], se[b])

        def wait_in(b):
            pltpu.make_async_copy(dummy_hbm, xb[b], sx[b]).wait()
            pltpu.make_async_copy(dummy_hbm, eb[b], se[b]).wait()

        def wait_out(b):
            pltpu.make_async_copy(ob[b], dummy_hbm, so[b]).wait()

        def compute(b):
            def add_body(i, _):
                base = i * (_LANES * _UNROLL)
                for u in range(_UNROLL):
                    sl = pl.ds(base + u * _LANES, _LANES)
                    ob[b][sl] = xb[b][sl] + eb[b][sl]
                return 0

            lax.fori_loop(0, slices_per_chunk, add_body, 0)

        def issue_out(k, b):
            row = base_row + k * C
            for r in range(C):
                src = pl.ds(r * d, d)
                pltpu.async_copy(ob[b].at[src], out_hbm.at[row + r], so[b])

        def step(k, b, first, last):
            wait_in(b)
            if not first:
                wait_out(b)
            compute(b)
            issue_out(k, b)
            if not last:
                issue_in(k + 2, b)

        issue_in(0, 0)
        issue_in(1, 1)
        step(0, 0, True, False)
        step(1, 1, True, False)

        def pair_body(i, _):
            k = 2 * i
            step(k, 0, False, False)
            step(k + 1, 1, False, False)
            return 0

        lax.fori_loop(1, n_chunks // 2 - 1, pair_body, 0)

        step(n_chunks - 2, 0, False, True)
        step(n_chunks - 1, 1, False, True)
        wait_out(0)
        wait_out(1)

    out2 = sc_add(x2, emb_weight, dummy)
    return out2.reshape(batch, seq_len, d)


# speedup vs baseline: 3.6655x; 1.2294x over previous
import functools

import jax
import jax.numpy as jnp
from jax import lax
from jax.experimental import pallas as pl
from jax.experimental.pallas import tpu as pltpu
from jax.experimental.pallas import tpu_sc as plsc

_NUM_WORKERS = 32
_CHUNK_ROWS = 8
_LANES = 16
_UNROLL = 4
_XDEPTH = 3
_EDEPTH = 2


def kernel(x, emb_weight):
    batch, seq_len, d = x.shape
    total_rows = batch * seq_len
    seq_per_w = seq_len // _NUM_WORKERS
    C = _CHUNK_ROWS
    n_chunks = seq_per_w // C
    chunk = C * d
    groups = chunk // (_LANES * _UNROLL)

    x2 = x.reshape(total_rows, d)
    dummy = jnp.zeros((chunk,), jnp.float32)

    mesh = plsc.VectorSubcoreMesh(core_axis_name="c", subcore_axis_name="s")

    vmem_types = [pltpu.VMEM((chunk,), jnp.float32)
                  for _ in range(_XDEPTH * batch + _EDEPTH)]
    sem_types = [pltpu.SemaphoreType.DMA
                 for _ in range(2 * _XDEPTH + _EDEPTH)]

    @functools.partial(
        pl.kernel,
        mesh=mesh,
        out_type=jax.ShapeDtypeStruct((total_rows, d), jnp.float32),
        scratch_types=vmem_types + sem_types,
    )
    def sc_add(x_hbm, emb_hbm, dummy_hbm, out_hbm, *scr):
        bufs = scr[:_XDEPTH * batch + _EDEPTH]
        sems = scr[_XDEPTH * batch + _EDEPTH:]
        xg = tuple(tuple(bufs[p * batch + j] for j in range(batch))
                   for p in range(_XDEPTH))
        eb = tuple(bufs[_XDEPTH * batch + e] for e in range(_EDEPTH))
        sx = sems[:_XDEPTH]
        so = sems[_XDEPTH:2 * _XDEPTH]
        se = sems[2 * _XDEPTH:]

        cid = lax.axis_index("c")
        sid = lax.axis_index("s")
        wid = sid * 2 + cid
        seq_base = wid * seq_per_w

        def issue_in(k, p, e):
            erow = seq_base + k * C

            def cp_body(r, _):
                dst = pl.ds(r * d, d)
                pltpu.async_copy(emb_hbm.at[erow + r], eb[e].at[dst], se[e])
                for j in range(batch):
                    pltpu.async_copy(x_hbm.at[j * seq_len + erow + r],
                                     xg[p][j].at[dst], sx[p])
                return 0

            lax.fori_loop(0, C, cp_body, 0)

        def wait_in(p, e):
            pltpu.make_async_copy(dummy_hbm, eb[e], se[e]).wait()
            for j in range(batch):
                pltpu.make_async_copy(dummy_hbm, xg[p][j], sx[p]).wait()

        def wait_out(p):
            for j in range(batch):
                pltpu.make_async_copy(xg[p][j], dummy_hbm, so[p]).wait()

        def compute(p, e):
            def add_body(i, _):
                base = i * (_LANES * _UNROLL)
                for u in range(_UNROLL):
                    sl = pl.ds(base + u * _LANES, _LANES)
                    ve = eb[e][sl]
                    for j in range(batch):
                        xg[p][j][sl] = xg[p][j][sl] + ve
                return 0

            lax.fori_loop(0, groups, add_body, 0)

        def issue_out(k, p):
            erow = seq_base + k * C

            def cp_body(r, _):
                src = pl.ds(r * d, d)
                for j in range(batch):
                    pltpu.async_copy(xg[p][j].at[src],
                                     out_hbm.at[j * seq_len + erow + r], so[p])
                return 0

            lax.fori_loop(0, C, cp_body, 0)

        def step(k, kmod3, kmod2, reuse_wait, issue_next):
            wait_in(kmod3, kmod2)
            compute(kmod3, kmod2)
            issue_out(k, kmod3)
            if issue_next:
                p_next = (kmod3 + 2) % _XDEPTH
                if reuse_wait:
                    wait_out(p_next)
                issue_in(k + 2, p_next, kmod2)

        issue_in(0, 0, 0)
        issue_in(1, 1, 1)
        step(0, 0, 0, False, True)
        step(1, 1, 1, True, True)

        def six_body(i, _):
            k0 = 2 + 6 * i
            for jj in range(6):
                step(k0 + jj, (2 + jj) % _XDEPTH, jj % _EDEPTH, True, True)
            return 0

        lax.fori_loop(0, (n_chunks - 4) // 6, six_body, 0)

        step(n_chunks - 2, (n_chunks - 2) % _XDEPTH, (n_chunks - 2) % _EDEPTH,
             False, False)
        step(n_chunks - 1, (n_chunks - 1) % _XDEPTH, (n_chunks - 1) % _EDEPTH,
             False, False)
        wait_out((n_chunks - 3) % _XDEPTH)
        wait_out((n_chunks - 2) % _XDEPTH)
        wait_out((n_chunks - 1) % _XDEPTH)

    out2 = sc_add(x2, emb_weight, dummy)
    return out2.reshape(batch, seq_len, d)
